# baseline (speedup 1.0000x reference)
import functools

import jax
import jax.numpy as jnp
from jax import lax
from jax.experimental import pallas as pl
from jax.experimental.pallas import tpu as pltpu
from jax.experimental.pallas import tpu_sc as plsc

VOCAB = 1000000
EMBED = 64
HIDDEN = 512
NUM_CLASSES = 128
BATCH = 4096
SEQ = 200

_LANES = 128
_XPAD = 256
_CA = 112
_CB = SEQ - _CA
_EG = EMBED // 16


def _make_sc_pool():
    info = plsc.get_sparse_core_info()
    nc, ns = info.num_cores, info.num_subcores
    nw = nc * ns
    rows_per_w = BATCH // nw

    mesh = plsc.VectorSubcoreMesh(core_axis_name="c", subcore_axis_name="s")

    @functools.partial(
        pl.kernel,
        mesh=mesh,
        out_type=jax.ShapeDtypeStruct((BATCH, _LANES), jnp.float32),
        scratch_types=[
            pltpu.VMEM((rows_per_w, _XPAD), jnp.int32),
            pltpu.VMEM((_XPAD,), jnp.int32),
            pltpu.VMEM((_XPAD,), jnp.int32),
            pltpu.VMEM((_CA, _LANES), jnp.float32),
            pltpu.VMEM((_CA, _LANES), jnp.float32),
            pltpu.VMEM((_CB, _LANES), jnp.float32),
            pltpu.VMEM((_CB, _LANES), jnp.float32),
            pltpu.VMEM((rows_per_w, _LANES), jnp.float32),
            pltpu.SemaphoreType.DMA,
            pltpu.SemaphoreType.DMA,
            pltpu.SemaphoreType.DMA,
            pltpu.SemaphoreType.DMA,
        ],
    )
    def sc_pool(x_hbm, table_hbm, out_hbm, idx_v, half0, half1,
                bufa0, bufa1, bufb0, bufb1, out_v,
                sema0, sema1, semb0, semb1):
        wid = lax.axis_index("s") * nc + lax.axis_index("c")
        row0 = wid * rows_per_w

        pltpu.sync_copy(x_hbm.at[pl.ds(row0, rows_per_w)], idx_v)

        zero = jnp.zeros((16,), jnp.float32)

        def zero_body(r, _):
            for g in range(4, 8):
                out_v[r, pl.ds(16 * g, 16)] = zero
            return 0
        lax.fori_loop(0, rows_per_w, zero_body, 0)

        def make_half(i, half):
            def body(k, _):
                half[pl.ds(16 * k, 16)] = (
                    idx_v[i, pl.ds(16 * k, 16)] >> 1)
                return 0
            lax.fori_loop(0, SEQ // 8 // 2 + 1, body, 0, unroll=4)

        def fire(half, buf, sem, off, n):
            pltpu.async_copy(table_hbm.at[half.at[pl.ds(off, n)]], buf, sem)

        def wait(buf, sem):
            pltpu.make_async_copy(
                table_hbm.at[half0.at[pl.ds(0, buf.shape[0])]], buf,
                sem).wait()

        def sum_rows(buf, pvec, b, nrows, acc):
            a = list(acc)
            for j in range(nrows):
                r = 16 * b + j
                par = pvec[j] == 1
                for g in range(_EG):
                    lo = buf[r, pl.ds(16 * g, 16)]
                    hi = buf[r, pl.ds(EMBED + 16 * g, 16)]
                    a[g] = a[g] + jnp.where(par, hi, lo)
            return tuple(a)

        def sum_chunk(i, base, buf, n, acc):
            def blk_body(b, a):
                pvec = idx_v[i, pl.ds(base + 16 * b, 16)] & 1
                return sum_rows(buf, pvec, b, 16, a)
            acc = lax.fori_loop(0, n // 16, blk_body, acc)
            if n % 16:
                nb = n // 16
                pvec = idx_v[i, pl.ds(base + 16 * nb, 16)] & 1
                acc = sum_rows(buf, pvec, nb, n % 16, acc)
            return acc

        make_half(0, half0)
        fire(half0, bufa0, sema0, 0, _CA)
        fire(half0, bufb0, semb0, _CA, _CB)

        def do_row(i, bufa, sema, bufb, semb, fire_next, nhalf, next_a,
                   next_sa, next_b, next_sb):
            @pl.when(fire_next)
            def _():
                make_half(i + 1, nhalf)
                fire(nhalf, next_a, next_sa, 0, _CA)
                fire(nhalf, next_b, next_sb, _CA, _CB)

            wait(bufa, sema)
            acc = sum_chunk(i, 0, bufa, _CA, (zero,) * _EG)
            wait(bufb, semb)
            acc = sum_chunk(i, _CA, bufb, _CB, acc)
            for g in range(_EG):
                out_v[i, pl.ds(16 * g, 16)] = acc[g]

        def pair_body(k, _):
            i = 2 * k
            do_row(i, bufa0, sema0, bufb0, semb0, True,
                   half1, bufa1, sema1, bufb1, semb1)
            do_row(i + 1, bufa1, sema1, bufb1, semb1,
                   i + 2 < rows_per_w, half0, bufa0, sema0, bufb0, semb0)
            return 0

        lax.fori_loop(0, rows_per_w // 2, pair_body, 0)

        pltpu.sync_copy(out_v, out_hbm.at[pl.ds(row0, rows_per_w)])

    return sc_pool


_sc_pool = None


def _mlp_body(p_ref, w1_ref, b1_ref, w2_ref, b2_ref, o_ref):
    p = p_ref[...] * (1.0 / SEQ)
    h = jnp.dot(p, w1_ref[...], preferred_element_type=jnp.float32)
    h = jnp.maximum(h + b1_ref[...], 0.0)
    o = jnp.dot(h, w2_ref[...], preferred_element_type=jnp.float32)
    o_ref[...] = o + b2_ref[...]


def _mlp(pooled, W1p, b1, W2, b2):
    blk = 512
    return pl.pallas_call(
        _mlp_body,
        grid=(BATCH // blk,),
        in_specs=[
            pl.BlockSpec((blk, _LANES), lambda i: (i, 0)),
            pl.BlockSpec((_LANES, HIDDEN), lambda i: (0, 0)),
            pl.BlockSpec((1, HIDDEN), lambda i: (0, 0)),
            pl.BlockSpec((HIDDEN, NUM_CLASSES), lambda i: (0, 0)),
            pl.BlockSpec((1, NUM_CLASSES), lambda i: (0, 0)),
        ],
        out_specs=pl.BlockSpec((blk, NUM_CLASSES), lambda i: (i, 0)),
        out_shape=jax.ShapeDtypeStruct((BATCH, NUM_CLASSES), jnp.float32),
    )(pooled, W1p, b1.reshape(1, HIDDEN), W2, b2.reshape(1, NUM_CLASSES))


def _prep_w1(W1):
    return jnp.concatenate(
        [W1, jnp.zeros((_LANES - EMBED, HIDDEN), W1.dtype)], axis=0)


def kernel(x, table, W1, b1, W2, b2):
    global _sc_pool
    if _sc_pool is None:
        _sc_pool = _make_sc_pool()
    t2 = table.reshape(VOCAB // 2, 2 * EMBED)
    x256 = jnp.pad(x.astype(jnp.int32), ((0, 0), (0, _XPAD - SEQ)))
    pooled = _sc_pool(x256, t2)
    return _mlp(pooled, _prep_w1(W1), b1, W2, b2)
